# Initial kernel scaffold; baseline (speedup 1.0000x reference)
import jax
import jax.numpy as jnp
from jax.experimental import pallas as pl


def kernel(observations, action_indices, W_obs, b_obs, W1, b1, W2, b2):
    raise NotImplementedError("write your pallas kernel here")


# speedup vs baseline: 1.7639x; 1.7639x over previous
import functools

import jax
import jax.numpy as jnp
from jax.experimental import pallas as pl

N = 64
OBS_DIM = 1024
LOWD = 512
K = 512
H = 8
A = 128
HID = 512


def _stage_a(obs_ref, w_obs_ref, b_obs_ref, w1_ref, b1_ref, idx_ref,
             obs_proj_ref, act_proj_ref):
    obs_lowd = jnp.dot(obs_ref[...], w_obs_ref[...],
                       preferred_element_type=jnp.float32) + b_obs_ref[...]
    obs_proj_ref[...] = jnp.dot(obs_lowd, w1_ref[:LOWD, :],
                                preferred_element_type=jnp.float32) + b1_ref[...]
    iota_a = jax.lax.broadcasted_iota(jnp.int32, (K, A), 1)
    acc = jnp.zeros((K, HID), dtype=jnp.float32)
    for h in range(H):
        onehot = (idx_ref[:, h:h + 1] == iota_a).astype(jnp.float32)
        acc = acc + jnp.dot(onehot, w1_ref[LOWD + h * A:LOWD + (h + 1) * A, :],
                            preferred_element_type=jnp.float32)
    act_proj_ref[...] = acc


def _stage_b(obs_proj_ref, act_proj_ref, w2_ref, b2_ref, idx0_ref,
             action_ref, value_ref):
    t = jnp.maximum(act_proj_ref[...] + obs_proj_ref[0], 0.0)
    out = jnp.dot(t, w2_ref[...], preferred_element_type=jnp.float32) + b2_ref[...]
    vals = out[:, :H]
    lg = out[:, H:]
    m = jnp.max(lg, axis=1, keepdims=True)
    e = jnp.exp(lg - m)
    sm = e / jnp.sum(e, axis=1, keepdims=True)
    v = jnp.sum(vals * sm, axis=1, keepdims=True)
    vmax = jnp.max(v)
    iota_k = jax.lax.broadcasted_iota(jnp.int32, (K, 1), 0)
    karg = jnp.min(jnp.where(v >= vmax, iota_k, K))
    aidx = jnp.sum(jnp.where(iota_k == karg, idx0_ref[...], 0))
    iota_a = jax.lax.broadcasted_iota(jnp.int32, (1, A), 1)
    action_ref[0] = (iota_a == aidx).astype(jnp.float32)
    value_ref[...] = jnp.reshape(vmax, (1, 1, 1))


@jax.jit
def kernel(observations, action_indices, W_obs, b_obs, W1, b1, W2, b2):
    idx = action_indices.reshape(K, H)
    obs_proj, act_proj = pl.pallas_call(
        _stage_a,
        out_shape=(
            jax.ShapeDtypeStruct((N, HID), jnp.float32),
            jax.ShapeDtypeStruct((K, HID), jnp.float32),
        ),
    )(observations, W_obs, b_obs.reshape(1, LOWD), W1, b1.reshape(1, HID), idx)

    action, value = pl.pallas_call(
        _stage_b,
        grid=(N,),
        in_specs=[
            pl.BlockSpec((1, 1, HID), lambda i: (i, 0, 0)),
            pl.BlockSpec((K, HID), lambda i: (0, 0)),
            pl.BlockSpec((HID, 2 * H), lambda i: (0, 0)),
            pl.BlockSpec((1, 2 * H), lambda i: (0, 0)),
            pl.BlockSpec((K, 1), lambda i: (0, 0)),
        ],
        out_specs=(
            pl.BlockSpec((1, 1, A), lambda i: (i, 0, 0)),
            pl.BlockSpec((1, 1, 1), lambda i: (i, 0, 0)),
        ),
        out_shape=(
            jax.ShapeDtypeStruct((N, 1, A), jnp.float32),
            jax.ShapeDtypeStruct((N, 1, 1), jnp.float32),
        ),
    )(obs_proj.reshape(N, 1, HID), act_proj, W2, b2.reshape(1, 2 * H), idx[:, 0:1])
    return (action.reshape(N, A), value.reshape(N))


# speedup vs baseline: 3.1095x; 1.7629x over previous
import functools

import jax
import jax.numpy as jnp
from jax.experimental import pallas as pl

N = 64
OBS_DIM = 1024
LOWD = 512
K = 512
H = 8
A = 128
HID = 512


def _stage_a(obs_ref, w_obs_ref, b_obs_ref, w1_ref, b1_ref, idx_ref,
             obs_proj_ref, act_proj_ref):
    obs_lowd = jnp.dot(obs_ref[...], w_obs_ref[...],
                       preferred_element_type=jnp.float32) + b_obs_ref[...]
    obs_proj_ref[...] = jnp.dot(obs_lowd, w1_ref[:LOWD, :],
                                preferred_element_type=jnp.float32) + b1_ref[...]
    iota_a = jax.lax.broadcasted_iota(jnp.int32, (K, A), 1)
    acc = jnp.zeros((K, HID), dtype=jnp.float32)
    for h in range(H):
        onehot = (idx_ref[:, h:h + 1] == iota_a).astype(jnp.float32)
        acc = acc + jnp.dot(onehot, w1_ref[LOWD + h * A:LOWD + (h + 1) * A, :],
                            preferred_element_type=jnp.float32)
    act_proj_ref[...] = acc


B = 8


def _stage_b(obs_proj_ref, act_proj_ref, w2_ref, b2_ref, idx0_ref,
             action_ref, value_ref):
    o = obs_proj_ref[...]
    t3 = jnp.maximum(o[:, None, :] + act_proj_ref[...][None, :, :], 0.0)
    t2 = t3.reshape(B * K, HID)
    out = jnp.dot(t2, w2_ref[...], preferred_element_type=jnp.float32) + b2_ref[...]
    vals = out[:, :H]
    lg = out[:, H:]
    m = jnp.max(lg, axis=1, keepdims=True)
    e = jnp.exp(lg - m)
    sm = e / jnp.sum(e, axis=1, keepdims=True)
    v3 = jnp.sum(vals * sm, axis=1, keepdims=True).reshape(B, K, 1)
    vmax = jnp.max(v3, axis=1, keepdims=True)
    iota_k = jax.lax.broadcasted_iota(jnp.int32, (B, K, 1), 1)
    karg = jnp.min(jnp.where(v3 >= vmax, iota_k, K), axis=1, keepdims=True)
    aidx = jnp.sum(jnp.where(iota_k == karg, idx0_ref[...][None, :, :], 0),
                   axis=1)
    iota_a = jax.lax.broadcasted_iota(jnp.int32, (B, A), 1)
    action_ref[...] = (iota_a == aidx).astype(jnp.float32)
    value_ref[...] = vmax.reshape(B, 1)


@jax.jit
def kernel(observations, action_indices, W_obs, b_obs, W1, b1, W2, b2):
    idx = action_indices.reshape(K, H)
    obs_proj, act_proj = pl.pallas_call(
        _stage_a,
        out_shape=(
            jax.ShapeDtypeStruct((N, HID), jnp.float32),
            jax.ShapeDtypeStruct((K, HID), jnp.float32),
        ),
    )(observations, W_obs, b_obs.reshape(1, LOWD), W1, b1.reshape(1, HID), idx)

    action, value = pl.pallas_call(
        _stage_b,
        grid=(N // B,),
        in_specs=[
            pl.BlockSpec((B, HID), lambda i: (i, 0)),
            pl.BlockSpec((K, HID), lambda i: (0, 0)),
            pl.BlockSpec((HID, 2 * H), lambda i: (0, 0)),
            pl.BlockSpec((1, 2 * H), lambda i: (0, 0)),
            pl.BlockSpec((K, 1), lambda i: (0, 0)),
        ],
        out_specs=(
            pl.BlockSpec((B, A), lambda i: (i, 0)),
            pl.BlockSpec((B, 1), lambda i: (i, 0)),
        ),
        out_shape=(
            jax.ShapeDtypeStruct((N, A), jnp.float32),
            jax.ShapeDtypeStruct((N, 1), jnp.float32),
        ),
    )(obs_proj, act_proj, W2, b2.reshape(1, 2 * H), idx[:, 0:1])
    return (action, value.reshape(N))


# speedup vs baseline: 5.7071x; 1.8354x over previous
import jax
import jax.numpy as jnp
from jax.experimental import pallas as pl

N = 64
OBS_DIM = 1024
LOWD = 512
K = 512
H = 8
A = 128
HID = 512
B = 8


def _stage_a(obs_ref, w_obs_ref, b_obs_ref, w1_ref, b1c_ref, idxT_ref,
             obs_projT_ref, act_projT_ref):
    obs_lowd = jnp.dot(obs_ref[...], w_obs_ref[...],
                       preferred_element_type=jnp.float32) + b_obs_ref[...]
    obs_projT_ref[...] = jax.lax.dot_general(
        w1_ref[:LOWD, :], obs_lowd, (((0,), (1,)), ((), ())),
        preferred_element_type=jnp.float32) + b1c_ref[...]
    iota_a = jax.lax.broadcasted_iota(jnp.int32, (A, K), 0)
    acc = jnp.zeros((HID, K), dtype=jnp.float32)
    for h in range(H):
        onehotT = (iota_a == idxT_ref[h:h + 1, :]).astype(jnp.float32)
        acc = acc + jax.lax.dot_general(
            w1_ref[LOWD + h * A:LOWD + (h + 1) * A, :], onehotT,
            (((0,), (0,)), ((), ())), preferred_element_type=jnp.float32)
    act_projT_ref[...] = acc


def _stage_b(o3_ref, actT_ref, w2T_ref, b2c_ref, idx0_ref,
             action_ref, value_ref):
    oT = o3_ref[0]
    tT = jnp.maximum(oT[:, :, None] + actT_ref[...][:, None, :],
                     0.0).reshape(HID, B * K)
    outT = jnp.dot(w2T_ref[...], tT,
                   preferred_element_type=jnp.float32) + b2c_ref[...]
    out3 = outT.reshape(2 * H, B, K)
    vals = out3[:H]
    lg = out3[H:]
    m = jnp.max(lg, axis=0, keepdims=True)
    e = jnp.exp(lg - m)
    s = jnp.sum(e, axis=0)
    v = jnp.sum(vals * e, axis=0) / s
    vmax = jnp.max(v, axis=1, keepdims=True)
    iota_k = jax.lax.broadcasted_iota(jnp.int32, (B, K), 1)
    karg = jnp.min(jnp.where(v >= vmax, iota_k, K), axis=1, keepdims=True)
    aidx = jnp.sum(jnp.where(iota_k == karg, idx0_ref[...], 0),
                   axis=1, keepdims=True)
    iota_a = jax.lax.broadcasted_iota(jnp.int32, (B, A), 1)
    action_ref[...] = (iota_a == aidx).astype(jnp.float32)
    value_ref[...] = vmax


@jax.jit
def kernel(observations, action_indices, W_obs, b_obs, W1, b1, W2, b2):
    idx = action_indices.reshape(K, H)
    obs_projT, act_projT = pl.pallas_call(
        _stage_a,
        out_shape=(
            jax.ShapeDtypeStruct((HID, N), jnp.float32),
            jax.ShapeDtypeStruct((HID, K), jnp.float32),
        ),
    )(observations, W_obs, b_obs.reshape(1, LOWD), W1, b1.reshape(HID, 1),
      idx.T)

    o3 = obs_projT.reshape(HID, N // B, B).transpose(1, 0, 2)
    action, value = pl.pallas_call(
        _stage_b,
        grid=(N // B,),
        in_specs=[
            pl.BlockSpec((1, HID, B), lambda i: (i, 0, 0)),
            pl.BlockSpec((HID, K), lambda i: (0, 0)),
            pl.BlockSpec((2 * H, HID), lambda i: (0, 0)),
            pl.BlockSpec((2 * H, 1), lambda i: (0, 0)),
            pl.BlockSpec((1, K), lambda i: (0, 0)),
        ],
        out_specs=(
            pl.BlockSpec((B, A), lambda i: (i, 0)),
            pl.BlockSpec((B, 1), lambda i: (i, 0)),
        ),
        out_shape=(
            jax.ShapeDtypeStruct((N, A), jnp.float32),
            jax.ShapeDtypeStruct((N, 1), jnp.float32),
        ),
    )(o3, act_projT, W2.T, b2.reshape(2 * H, 1), idx[:, 0].reshape(1, K))
    return (action, value.reshape(N))


# speedup vs baseline: 9.6499x; 1.6909x over previous
import jax
import jax.numpy as jnp
from jax.experimental import pallas as pl
from jax.experimental.pallas import tpu as pltpu

N = 64
OBS_DIM = 1024
LOWD = 512
K = 512
H = 8
A = 128
HID = 512
B = 8


def _fused(obs_ref, w_obs_ref, b_obs_ref, w1_ref, b1c_ref, idxT_ref,
           w2T_ref, b2c_ref, idx0_ref, action_ref, value_ref,
           act_projT_s):
    i = pl.program_id(0)

    @pl.when(i == 0)
    def _stage_a():
        iota_a = jax.lax.broadcasted_iota(jnp.int32, (A, K), 0)
        acc = jnp.zeros((HID, K), dtype=jnp.float32)
        for h in range(H):
            onehotT = (iota_a == idxT_ref[h:h + 1, :]).astype(jnp.float32)
            acc = acc + jax.lax.dot_general(
                w1_ref[LOWD + h * A:LOWD + (h + 1) * A, :], onehotT,
                (((0,), (0,)), ((), ())), preferred_element_type=jnp.float32)
        act_projT_s[...] = acc

    obs_lowd = jnp.dot(obs_ref[...], w_obs_ref[...],
                       preferred_element_type=jnp.float32) + b_obs_ref[...]
    oT8 = jax.lax.dot_general(
        w1_ref[:LOWD, :], obs_lowd, (((0,), (1,)), ((), ())),
        preferred_element_type=jnp.float32) + b1c_ref[...]
    actT = act_projT_s[...]
    w2T = w2T_ref[...]
    slabs = []
    for b in range(B):
        tb = jnp.maximum(actT + oT8[:, b:b + 1], 0.0)
        slabs.append(jax.lax.dot_general(
            w2T, tb, (((1,), (0,)), ((), ())),
            preferred_element_type=jnp.float32))
    out3 = jnp.stack(slabs, axis=1) + b2c_ref[...][:, :, None]
    vals = out3[:H]
    lg = out3[H:]
    m = jnp.max(lg, axis=0, keepdims=True)
    e = jnp.exp(lg - m)
    s = jnp.sum(e, axis=0)
    v = jnp.sum(vals * e, axis=0) / s
    vmax = jnp.max(v, axis=1, keepdims=True)
    iota_k = jax.lax.broadcasted_iota(jnp.int32, (B, K), 1)
    karg = jnp.min(jnp.where(v >= vmax, iota_k, K), axis=1, keepdims=True)
    aidx = jnp.sum(jnp.where(iota_k == karg, idx0_ref[...], 0),
                   axis=1, keepdims=True)
    iota_a = jax.lax.broadcasted_iota(jnp.int32, (B, A), 1)
    action_ref[...] = (iota_a == aidx).astype(jnp.float32)
    value_ref[...] = vmax


@jax.jit
def kernel(observations, action_indices, W_obs, b_obs, W1, b1, W2, b2):
    idx = action_indices.reshape(K, H)
    action, value = pl.pallas_call(
        _fused,
        grid=(N // B,),
        in_specs=[
            pl.BlockSpec((B, OBS_DIM), lambda i: (i, 0)),
            pl.BlockSpec((OBS_DIM, LOWD), lambda i: (0, 0)),
            pl.BlockSpec((1, LOWD), lambda i: (0, 0)),
            pl.BlockSpec((LOWD + H * A, HID), lambda i: (0, 0)),
            pl.BlockSpec((HID, 1), lambda i: (0, 0)),
            pl.BlockSpec((H, K), lambda i: (0, 0)),
            pl.BlockSpec((2 * H, HID), lambda i: (0, 0)),
            pl.BlockSpec((2 * H, 1), lambda i: (0, 0)),
            pl.BlockSpec((1, K), lambda i: (0, 0)),
        ],
        out_specs=(
            pl.BlockSpec((B, A), lambda i: (i, 0)),
            pl.BlockSpec((B, 1), lambda i: (i, 0)),
        ),
        out_shape=(
            jax.ShapeDtypeStruct((N, A), jnp.float32),
            jax.ShapeDtypeStruct((N, 1), jnp.float32),
        ),
        scratch_shapes=[
            pltpu.VMEM((HID, K), jnp.float32),
        ],
    )(observations, W_obs, b_obs.reshape(1, LOWD), W1, b1.reshape(HID, 1),
      idx.T, W2.T, b2.reshape(2 * H, 1), idx[:, 0].reshape(1, K))
    return (action, value.reshape(N))


# speedup vs baseline: 12.2774x; 1.2723x over previous
import jax
import jax.numpy as jnp
from jax.experimental import pallas as pl

N = 64
OBS_DIM = 1024
LOWD = 512
K = 512
H = 8
A = 128
HID = 512


def _fused(obs_ref, w_obs_ref, b_obs_ref, w1_ref, b1c_ref, idxT_ref,
           w2T_ref, b2c_ref, idx0_ref, action_ref, value_ref):
    iota_a = jax.lax.broadcasted_iota(jnp.int32, (A, K), 0)
    actT = jnp.zeros((HID, K), dtype=jnp.float32)
    for h in range(H):
        onehotT = (iota_a == idxT_ref[h:h + 1, :]).astype(jnp.float32)
        actT = actT + jax.lax.dot_general(
            w1_ref[LOWD + h * A:LOWD + (h + 1) * A, :], onehotT,
            (((0,), (0,)), ((), ())), preferred_element_type=jnp.float32)

    obs_lowd = jnp.dot(obs_ref[...], w_obs_ref[...],
                       preferred_element_type=jnp.float32) + b_obs_ref[...]
    oT = jax.lax.dot_general(
        w1_ref[:LOWD, :], obs_lowd, (((0,), (1,)), ((), ())),
        preferred_element_type=jnp.float32) + b1c_ref[...]
    w2T = w2T_ref[...]
    slabs = []
    for b in range(N):
        tb = jnp.maximum(actT + oT[:, b:b + 1], 0.0)
        slabs.append(jax.lax.dot_general(
            w2T, tb, (((1,), (0,)), ((), ())),
            preferred_element_type=jnp.float32))
    out3 = jnp.stack(slabs, axis=1) + b2c_ref[...][:, :, None]
    vals = out3[:H]
    lg = out3[H:]
    m = jnp.max(lg, axis=0, keepdims=True)
    e = jnp.exp(lg - m)
    s = jnp.sum(e, axis=0)
    v = jnp.sum(vals * e, axis=0) / s
    vmax = jnp.max(v, axis=1, keepdims=True)
    iota_k = jax.lax.broadcasted_iota(jnp.int32, (N, K), 1)
    karg = jnp.min(jnp.where(v >= vmax, iota_k, K), axis=1, keepdims=True)
    aidx = jnp.sum(jnp.where(iota_k == karg, idx0_ref[...], 0),
                   axis=1, keepdims=True)
    iota_act = jax.lax.broadcasted_iota(jnp.int32, (N, A), 1)
    action_ref[...] = (iota_act == aidx).astype(jnp.float32)
    value_ref[...] = vmax


@jax.jit
def kernel(observations, action_indices, W_obs, b_obs, W1, b1, W2, b2):
    idx = action_indices.reshape(K, H)
    action, value = pl.pallas_call(
        _fused,
        out_shape=(
            jax.ShapeDtypeStruct((N, A), jnp.float32),
            jax.ShapeDtypeStruct((N, 1), jnp.float32),
        ),
    )(observations, W_obs, b_obs.reshape(1, LOWD), W1, b1.reshape(HID, 1),
      idx.T, W2.T, b2.reshape(2 * H, 1), idx[:, 0].reshape(1, K))
    return (action, value.reshape(N))
